# baseline (speedup 1.0000x reference)
import functools

import jax
import jax.numpy as jnp
from jax import lax
from jax.experimental import pallas as pl
from jax.experimental.pallas import tpu as pltpu
from jax.experimental.pallas import tpu_sc as plsc

NUM_EMBEDDINGS = 1000000
D = 64
MD = 128
BATCH = 4096
HIST = 50
B_TOTAL = BATCH * HIST

NC, NS = 2, 16
NW = NC * NS
B_PER_W = B_TOTAL // NW
CHUNK = 128
NCH = B_PER_W // CHUNK


def _gather_body(idx_hbm, table_hbm, out_hbm, idx_v, rows_v, sem_g):
    wid = lax.axis_index("s") * NC + lax.axis_index("c")
    base = wid * B_PER_W
    pltpu.sync_copy(idx_hbm.at[wid], idx_v)

    def body(j, carry):
        pltpu.async_copy(table_hbm.at[idx_v.at[j]], rows_v, sem_g).wait()
        pltpu.sync_copy(rows_v, out_hbm.at[pl.ds(base + j * CHUNK, CHUNK)])
        return carry

    lax.fori_loop(0, NCH, body, 0)


_gather = pl.kernel(
    _gather_body,
    out_type=jax.ShapeDtypeStruct((B_TOTAL, D), jnp.float32),
    mesh=plsc.VectorSubcoreMesh(
        core_axis_name="c", subcore_axis_name="s", num_cores=NC, num_subcores=NS
    ),
    scratch_types=[
        pltpu.VMEM((NCH, CHUNK), jnp.int32),
        pltpu.VMEM((CHUNK, D), jnp.float32),
        pltpu.SemaphoreType.DMA,
    ],
    compiler_params=pltpu.CompilerParams(use_tc_tiling_on_sc=False),
)


def _mm_body(emb_ref, w_ref, out_ref):
    out_ref[...] = lax.dot_general(
        emb_ref[...], w_ref[...], (((1,), (1,)), ((), ())),
        preferred_element_type=jnp.float32,
    )


_BM = 2048
_mm = pl.pallas_call(
    _mm_body,
    grid=(B_TOTAL // _BM,),
    in_specs=[
        pl.BlockSpec((_BM, D), lambda i: (i, 0)),
        pl.BlockSpec((MD, D), lambda i: (0, 0)),
    ],
    out_specs=pl.BlockSpec((_BM, MD), lambda i: (i, 0)),
    out_shape=jax.ShapeDtypeStruct((B_TOTAL, MD), jnp.float32),
)


def kernel(input, table, W):
    idx = input.reshape(NW, NCH, CHUNK).astype(jnp.int32)
    emb = _gather(idx, table)
    out = _mm(emb, W)
    return out.reshape(BATCH, HIST, MD)


# speedup vs baseline: 2.2383x; 2.2383x over previous
import jax
import jax.numpy as jnp
from jax import lax
from jax.experimental import pallas as pl
from jax.experimental.pallas import tpu as pltpu
from jax.experimental.pallas import tpu_sc as plsc

TBL = 1000000
D = 64
MD = 128
BATCH = 4096
HIST = 50
B_TOTAL = BATCH * HIST

NC, NS = 2, 16
NW = NC * NS
B_PER_W = B_TOTAL // NW
CHUNK = 128
NCH = B_PER_W // CHUNK


def _proj_body(t_ref, w_ref, p_ref):
    p_ref[...] = lax.dot_general(
        t_ref[...], w_ref[...], (((0,), (0,)), ((), ())),
        preferred_element_type=jnp.float32,
    )


_NCOL = 4096
_proj = pl.pallas_call(
    _proj_body,
    grid=(pl.cdiv(TBL, _NCOL),),
    in_specs=[
        pl.BlockSpec((D, _NCOL), lambda i: (0, i)),
        pl.BlockSpec((D, MD), lambda i: (0, 0)),
    ],
    out_specs=pl.BlockSpec((_NCOL, MD), lambda i: (i, 0)),
    out_shape=jax.ShapeDtypeStruct((TBL, MD), jnp.float32),
)


def _gather_body(idx_hbm, p_hbm, out_hbm, idx_v, rows_v, sem_g):
    wid = lax.axis_index("s") * NC + lax.axis_index("c")
    base = wid * B_PER_W
    pltpu.sync_copy(idx_hbm.at[wid], idx_v)

    def body(j, carry):
        pltpu.async_copy(p_hbm.at[idx_v.at[j]], rows_v, sem_g).wait()
        pltpu.sync_copy(rows_v, out_hbm.at[pl.ds(base + j * CHUNK, CHUNK)])
        return carry

    lax.fori_loop(0, NCH, body, 0)


_gather = pl.kernel(
    _gather_body,
    out_type=jax.ShapeDtypeStruct((B_TOTAL, MD), jnp.float32),
    mesh=plsc.VectorSubcoreMesh(
        core_axis_name="c", subcore_axis_name="s", num_cores=NC, num_subcores=NS
    ),
    scratch_types=[
        pltpu.VMEM((NCH, CHUNK), jnp.int32),
        pltpu.VMEM((CHUNK, MD), jnp.float32),
        pltpu.SemaphoreType.DMA,
    ],
    compiler_params=pltpu.CompilerParams(use_tc_tiling_on_sc=True),
)


def kernel(input, table, W):
    p = _proj(table.T, W.T)
    idx = jnp.transpose(input).reshape(NW, NCH, CHUNK).astype(jnp.int32)
    out_flat = _gather(idx, p)
    return jnp.transpose(out_flat.reshape(HIST, BATCH, MD), (1, 0, 2))


# speedup vs baseline: 2.7994x; 1.2507x over previous
import jax
import jax.numpy as jnp
from jax import lax
from jax.experimental import pallas as pl
from jax.experimental.pallas import tpu as pltpu
from jax.experimental.pallas import tpu_sc as plsc

TBL = 1000000
D = 64
MD = 128
BATCH = 4096
HIST = 50
B_TOTAL = BATCH * HIST

NC, NS = 2, 16
NW = NC * NS
B_PER_W = B_TOTAL // NW
CHUNK = 128
NCH = B_PER_W // CHUNK


def _proj_body(t_ref, w_ref, p_ref):
    p_ref[...] = lax.dot_general(
        t_ref[...].astype(jnp.bfloat16), w_ref[...].astype(jnp.bfloat16),
        (((0,), (0,)), ((), ())),
        preferred_element_type=jnp.float32,
    )


_NCOL = 8192
_proj = pl.pallas_call(
    _proj_body,
    grid=(pl.cdiv(TBL, _NCOL),),
    in_specs=[
        pl.BlockSpec((D, _NCOL), lambda i: (0, i)),
        pl.BlockSpec((D, MD), lambda i: (0, 0)),
    ],
    out_specs=pl.BlockSpec((_NCOL, MD), lambda i: (i, 0)),
    out_shape=jax.ShapeDtypeStruct((TBL, MD), jnp.float32),
)


def _gather_body(idx_hbm, p_hbm, out_hbm, idx_v, rows_v, sem_g):
    wid = lax.axis_index("s") * NC + lax.axis_index("c")
    base = wid * B_PER_W
    pltpu.sync_copy(idx_hbm.at[wid], idx_v)

    def body(j, carry):
        pltpu.async_copy(p_hbm.at[idx_v.at[j]], rows_v, sem_g).wait()
        pltpu.sync_copy(rows_v, out_hbm.at[pl.ds(base + j * CHUNK, CHUNK)])
        return carry

    lax.fori_loop(0, NCH, body, 0)


_gather = pl.kernel(
    _gather_body,
    out_type=jax.ShapeDtypeStruct((B_TOTAL, MD), jnp.float32),
    mesh=plsc.VectorSubcoreMesh(
        core_axis_name="c", subcore_axis_name="s", num_cores=NC, num_subcores=NS
    ),
    scratch_types=[
        pltpu.VMEM((NCH, CHUNK), jnp.int32),
        pltpu.VMEM((CHUNK, MD), jnp.float32),
        pltpu.SemaphoreType.DMA,
    ],
    compiler_params=pltpu.CompilerParams(use_tc_tiling_on_sc=True),
)


def kernel(input, table, W):
    p = _proj(table.T, W.T)
    idx = jnp.transpose(input).reshape(NW, NCH, CHUNK).astype(jnp.int32)
    out_flat = _gather(idx, p)
    return jnp.transpose(out_flat.reshape(HIST, BATCH, MD), (1, 0, 2))


# speedup vs baseline: 3.1000x; 1.1074x over previous
import jax
import jax.numpy as jnp
from jax import lax
from jax.experimental import pallas as pl
from jax.experimental.pallas import tpu as pltpu
from jax.experimental.pallas import tpu_sc as plsc

TBL = 1000000
D = 64
MD = 128
BATCH = 4096
HIST = 50
B_TOTAL = BATCH * HIST

NC, NS = 2, 16
NW = NC * NS
B_PER_W = B_TOTAL // NW
CHUNK = 128
NCH = B_PER_W // CHUNK


def _proj_body(t_ref, w_ref, p_ref):
    p_ref[...] = lax.dot_general(
        t_ref[...].astype(jnp.bfloat16), w_ref[...].astype(jnp.bfloat16),
        (((0,), (0,)), ((), ())),
        preferred_element_type=jnp.float32,
    )


_NCOL = 8192
_proj = pl.pallas_call(
    _proj_body,
    grid=(pl.cdiv(TBL, _NCOL),),
    in_specs=[
        pl.BlockSpec((D, _NCOL), lambda i: (0, i)),
        pl.BlockSpec((D, MD), lambda i: (0, 0)),
    ],
    out_specs=pl.BlockSpec((_NCOL, MD), lambda i: (i, 0)),
    out_shape=jax.ShapeDtypeStruct((TBL, MD), jnp.float32),
)


def _gather_body(idx_hbm, p_hbm, out_hbm, idx_v, rows_v, sem_g, sem_w):
    wid = lax.axis_index("s") * NC + lax.axis_index("c")
    base = wid * B_PER_W
    pltpu.sync_copy(idx_hbm.at[wid], idx_v)
    pltpu.async_copy(p_hbm.at[idx_v.at[0]], rows_v.at[0], sem_g)
    pltpu.async_copy(p_hbm.at[idx_v.at[1]], rows_v.at[1], sem_g)

    def body(j, carry):
        @pl.when(j >= 2)
        def _():
            pltpu.make_async_copy(
                rows_v.at[0], out_hbm.at[pl.ds(base, CHUNK)], sem_w
            ).wait()

        @pl.when(j + 2 < NCH)
        def _():
            pltpu.async_copy(
                p_hbm.at[idx_v.at[j + 2]], rows_v.at[(j + 2) % 4], sem_g
            )

        pltpu.make_async_copy(
            p_hbm.at[idx_v.at[0]], rows_v.at[0], sem_g
        ).wait()
        pltpu.async_copy(
            rows_v.at[j % 4], out_hbm.at[pl.ds(base + j * CHUNK, CHUNK)], sem_w
        )
        return carry

    lax.fori_loop(0, NCH, body, 0)
    pltpu.make_async_copy(rows_v.at[0], out_hbm.at[pl.ds(base, CHUNK)], sem_w).wait()
    pltpu.make_async_copy(rows_v.at[0], out_hbm.at[pl.ds(base, CHUNK)], sem_w).wait()


_gather = pl.kernel(
    _gather_body,
    out_type=jax.ShapeDtypeStruct((B_TOTAL, MD), jnp.float32),
    mesh=plsc.VectorSubcoreMesh(
        core_axis_name="c", subcore_axis_name="s", num_cores=NC, num_subcores=NS
    ),
    scratch_types=[
        pltpu.VMEM((NCH, CHUNK), jnp.int32),
        pltpu.VMEM((4, CHUNK, MD), jnp.float32),
        pltpu.SemaphoreType.DMA,
        pltpu.SemaphoreType.DMA,
    ],
    compiler_params=pltpu.CompilerParams(use_tc_tiling_on_sc=True),
)


def kernel(input, table, W):
    p = _proj(table.T, W.T)
    idx = jnp.transpose(input).reshape(NW, NCH, CHUNK).astype(jnp.int32)
    out_flat = _gather(idx, p)
    return jnp.transpose(out_flat.reshape(HIST, BATCH, MD), (1, 0, 2))


# speedup vs baseline: 3.2045x; 1.0337x over previous
import jax
import jax.numpy as jnp
from jax import lax
from jax.experimental import pallas as pl
from jax.experimental.pallas import tpu as pltpu
from jax.experimental.pallas import tpu_sc as plsc

TBL = 1000000
D = 64
MD = 128
BATCH = 4096
HIST = 50
B_TOTAL = BATCH * HIST

NC, NS = 2, 16
NW = NC * NS
B_PER_W = B_TOTAL // NW
CHUNK = 128
NCH = B_PER_W // CHUNK


def _proj_body(t_ref, w_ref, p_ref):
    p_ref[...] = lax.dot_general(
        t_ref[...].astype(jnp.bfloat16), w_ref[...].astype(jnp.bfloat16),
        (((0,), (0,)), ((), ())),
        preferred_element_type=jnp.float32,
    )


_NCOL = 16384
_proj = pl.pallas_call(
    _proj_body,
    grid=(pl.cdiv(TBL, _NCOL),),
    in_specs=[
        pl.BlockSpec((D, _NCOL), lambda i: (0, i)),
        pl.BlockSpec((D, MD), lambda i: (0, 0)),
    ],
    out_specs=pl.BlockSpec((_NCOL, MD), lambda i: (i, 0)),
    out_shape=jax.ShapeDtypeStruct((TBL, MD), jnp.float32),
)


def _gather_body(idx_hbm, p_hbm, out_hbm, idx_v, rows_v, sem_g, sem_w):
    wid = lax.axis_index("s") * NC + lax.axis_index("c")
    base = wid * B_PER_W
    pltpu.sync_copy(idx_hbm.at[wid], idx_v)
    pltpu.async_copy(p_hbm.at[idx_v.at[0]], rows_v.at[0], sem_g)
    pltpu.async_copy(p_hbm.at[idx_v.at[1]], rows_v.at[1], sem_g)

    def body(j, carry):
        @pl.when(j >= 2)
        def _():
            pltpu.make_async_copy(
                rows_v.at[0], out_hbm.at[pl.ds(base, CHUNK)], sem_w
            ).wait()

        @pl.when(j + 2 < NCH)
        def _():
            pltpu.async_copy(
                p_hbm.at[idx_v.at[j + 2]], rows_v.at[(j + 2) % 4], sem_g
            )

        pltpu.make_async_copy(
            p_hbm.at[idx_v.at[0]], rows_v.at[0], sem_g
        ).wait()
        pltpu.async_copy(
            rows_v.at[j % 4], out_hbm.at[pl.ds(base + j * CHUNK, CHUNK)], sem_w
        )
        return carry

    lax.fori_loop(0, NCH, body, 0)
    pltpu.make_async_copy(rows_v.at[0], out_hbm.at[pl.ds(base, CHUNK)], sem_w).wait()
    pltpu.make_async_copy(rows_v.at[0], out_hbm.at[pl.ds(base, CHUNK)], sem_w).wait()


_gather = pl.kernel(
    _gather_body,
    out_type=jax.ShapeDtypeStruct((B_TOTAL, MD), jnp.float32),
    mesh=plsc.VectorSubcoreMesh(
        core_axis_name="c", subcore_axis_name="s", num_cores=NC, num_subcores=NS
    ),
    scratch_types=[
        pltpu.VMEM((NCH, CHUNK), jnp.int32),
        pltpu.VMEM((4, CHUNK, MD), jnp.float32),
        pltpu.SemaphoreType.DMA,
        pltpu.SemaphoreType.DMA,
    ],
    compiler_params=pltpu.CompilerParams(use_tc_tiling_on_sc=True),
)


def kernel(input, table, W):
    p = _proj(table.T, W.T)
    idx = jnp.transpose(input).reshape(NW, NCH, CHUNK).astype(jnp.int32)
    out_flat = _gather(idx, p)
    return jnp.transpose(out_flat.reshape(HIST, BATCH, MD), (1, 0, 2))
